# baseline (speedup 1.0000x reference)
import jax
import jax.numpy as jnp
from jax.experimental import pallas as pl
from jax.experimental.pallas import tpu as pltpu
from jax.experimental.pallas import tpu_sc as plsc

_K = 8192
_D = 256
_N = 8192
_TM = 512
_TN = 512
_SUB = 8
_GATHER_W = 128


def _argmin_body(z_ref, cb_ref, zn_ref, cn_ref, idx_ref, dsum_ref):
    nblk = _TN // _SUB
    zn = zn_ref[...]

    def chunk_mm(c):
        cb = cb_ref[pl.ds(c * _TN, _TN), :]
        return jax.lax.dot_general(
            cb, z_ref[...], (((1,), (1,)), ((), ())),
            preferred_element_type=jnp.float32)

    def consume(c, mmt, acc, rid):
        cnc = cn_ref[pl.ds(c * _TN, _TN), :]
        for i in range(nblk):
            cni = jax.lax.slice(cnc, (i * _SUB, 0), ((i + 1) * _SUB, 1))
            mmi = jax.lax.slice(mmt, (i * _SUB, 0), ((i + 1) * _SUB, _TM))
            d = (zn + cni) - mmi
            blkid = (c * nblk + i).astype(jnp.float32)
            better = d < acc
            acc = jnp.where(better, d, acc)
            rid = jnp.where(better, blkid, rid)
        return acc, rid

    _PIPE = 16

    def step(jj, carry):
        acc, rid = carry
        ca = jj * _PIPE
        mms = [chunk_mm(ca + p) for p in range(_PIPE)]
        for p in range(_PIPE):
            acc, rid = consume(ca + p, mms[p], acc, rid)
        return acc, rid

    acc0 = jnp.full((_SUB, _TM), jnp.inf, dtype=jnp.float32)
    rid0 = jnp.zeros((_SUB, _TM), dtype=jnp.float32)
    acc, rid = jax.lax.fori_loop(0, _K // _TN // _PIPE, step, (acc0, rid0))

    sub = jax.lax.broadcasted_iota(
        jnp.int32, (_SUB, _TM), 0).astype(jnp.float32)
    absidx = rid * jnp.float32(_SUB) + sub
    gmin = jnp.min(acc, axis=0, keepdims=True)
    cand = jnp.where(acc == gmin, absidx, jnp.float32(_K))
    gidx = jnp.min(cand, axis=0, keepdims=True)

    idx_ref[...] = gidx.astype(jnp.int32).reshape(1, 1, _TM)

    @pl.when(pl.program_id(0) == 0)
    def _():
        dsum_ref[...] = jnp.zeros((1, 1), jnp.float32)
    dsum_ref[...] += jnp.sum(gmin).reshape(1, 1)


def _run_argmin(z_flat, codebook, zn_row, cn_col):
    n = z_flat.shape[0]
    grid = (n // _TM,)
    return pl.pallas_call(
        _argmin_body,
        grid=grid,
        in_specs=[
            pl.BlockSpec((_TM, _D), lambda i: (i, 0)),
            pl.BlockSpec((_K, _D), lambda i: (0, 0)),
            pl.BlockSpec((1, _TM), lambda i: (0, i)),
            pl.BlockSpec((_K, 1), lambda i: (0, 0)),
        ],
        out_specs=[
            pl.BlockSpec((1, 1, _TM), lambda i: (i, 0, 0)),
            pl.BlockSpec((1, 1), lambda i: (0, 0)),
        ],
        out_shape=[
            jax.ShapeDtypeStruct((n // _TM, 1, _TM), jnp.int32),
            jax.ShapeDtypeStruct((1, 1), jnp.float32),
        ],
    )(z_flat, codebook, zn_row, cn_col)


def _sc_gather(codebook, idx_row):
    mesh = plsc.VectorSubcoreMesh(core_axis_name="core",
                                  subcore_axis_name="subcore")

    n = idx_row.shape[1]

    @pl.kernel(out_type=jax.ShapeDtypeStruct((n, _D), jnp.float32),
               mesh=mesh)
    def gather_kernel(cb_hbm, i_hbm, o_hbm):
        def body(i_vmem, o_vmem):
            pltpu.sync_copy(cb_hbm.at[i_vmem.at[0]], o_vmem)

        pltpu.emit_pipeline(
            body,
            grid=(n // _GATHER_W,),
            in_specs=[pl.BlockSpec((1, _GATHER_W), lambda i: (0, i))],
            out_specs=[pl.BlockSpec((_GATHER_W, _D), lambda i: (i, 0))],
            core_axis_name=("core", "subcore"),
            dimension_semantics=(pltpu.PARALLEL,),
        )(i_hbm, o_hbm)

    return gather_kernel(codebook, idx_row)


def kernel(z, codebook):
    B, C, H, W = z.shape
    z_p = jnp.transpose(z, (0, 2, 3, 1))
    z_flat = z_p.reshape(-1, _D)
    zn = jnp.sum(z_flat ** 2, axis=1, keepdims=True)
    cn = jnp.sum(codebook ** 2, axis=1, keepdims=True)

    z2 = z_flat * jnp.float32(2.0)
    zn_row = zn.T

    half = _N // 2
    hb = B // 2
    idx_a, dsum_a = _run_argmin(z2[:half], codebook, zn_row[:, :half], cn)
    idx_row_a = idx_a.reshape(1, half)
    zq_a = _sc_gather(codebook, idx_row_a)
    idx_b, dsum_b = _run_argmin(z2[half:], codebook, zn_row[:, half:], cn)
    idx_row_b = idx_b.reshape(1, half)
    zq_b = _sc_gather(codebook, idx_row_b)
    q_a = jnp.transpose(zq_a.reshape(hb, H, W, C), (0, 3, 1, 2))
    q_b = jnp.transpose(zq_b.reshape(hb, H, W, C), (0, 3, 1, 2))

    m = (dsum_a[0, 0] + dsum_b[0, 0]) / jnp.float32(_N * _D)
    codebook_loss = 0.25 * m + m

    quantized = jnp.concatenate([q_a, q_b], axis=0)
    indices = jnp.concatenate([idx_row_a, idx_row_b], axis=1).reshape(B, H, W)
    return (quantized, indices, codebook_loss)


# speedup vs baseline: 1.2038x; 1.2038x over previous
import jax
import jax.numpy as jnp
from jax.experimental import pallas as pl
from jax.experimental.pallas import tpu as pltpu
from jax.experimental.pallas import tpu_sc as plsc

_K = 8192
_D = 256
_N = 8192
_TM = 512
_TN = 512
_SUB = 8
_GATHER_W = 128


def _argmin_body(z_ref, cb_ref, zn_ref, cn_ref, idx_ref, dsum_ref):
    nblk = _TN // _SUB
    zn = zn_ref[...]

    def chunk_mm(c):
        cb = cb_ref[pl.ds(c * _TN, _TN), :]
        return jax.lax.dot_general(
            cb, z_ref[...], (((1,), (1,)), ((), ())),
            preferred_element_type=jnp.float32)

    def consume(c, mmt, acc, rid):
        cnc = cn_ref[pl.ds(c * _TN, _TN), :]
        for i in range(nblk):
            cni = jax.lax.slice(cnc, (i * _SUB, 0), ((i + 1) * _SUB, 1))
            mmi = jax.lax.slice(mmt, (i * _SUB, 0), ((i + 1) * _SUB, _TM))
            d = (zn + cni) - mmi
            blkid = (c * nblk + i).astype(jnp.float32)
            better = d < acc
            acc = jnp.where(better, d, acc)
            rid = jnp.where(better, blkid, rid)
        return acc, rid

    _PIPE = 16

    def step(jj, carry):
        acc, rid = carry
        ca = jj * _PIPE
        mms = [chunk_mm(ca + p) for p in range(_PIPE)]
        for p in range(_PIPE):
            acc, rid = consume(ca + p, mms[p], acc, rid)
        return acc, rid

    acc0 = jnp.full((_SUB, _TM), jnp.inf, dtype=jnp.float32)
    rid0 = jnp.zeros((_SUB, _TM), dtype=jnp.float32)
    acc, rid = jax.lax.fori_loop(0, _K // _TN // _PIPE, step, (acc0, rid0))

    sub = jax.lax.broadcasted_iota(
        jnp.int32, (_SUB, _TM), 0).astype(jnp.float32)
    absidx = rid * jnp.float32(_SUB) + sub
    gmin = jnp.min(acc, axis=0, keepdims=True)
    cand = jnp.where(acc == gmin, absidx, jnp.float32(_K))
    gidx = jnp.min(cand, axis=0, keepdims=True)

    idx_ref[...] = gidx.astype(jnp.int32).reshape(1, 1, _TM)

    @pl.when(pl.program_id(0) == 0)
    def _():
        dsum_ref[...] = jnp.zeros((1, 1), jnp.float32)
    dsum_ref[...] += jnp.sum(gmin).reshape(1, 1)


def _run_argmin(z_flat, codebook, zn_row, cn_col):
    grid = (_N // _TM,)
    return pl.pallas_call(
        _argmin_body,
        grid=grid,
        in_specs=[
            pl.BlockSpec((_TM, _D), lambda i: (i, 0)),
            pl.BlockSpec((_K, _D), lambda i: (0, 0)),
            pl.BlockSpec((1, _TM), lambda i: (0, i)),
            pl.BlockSpec((_K, 1), lambda i: (0, 0)),
        ],
        out_specs=[
            pl.BlockSpec((1, 1, _TM), lambda i: (i, 0, 0)),
            pl.BlockSpec((1, 1), lambda i: (0, 0)),
        ],
        out_shape=[
            jax.ShapeDtypeStruct((_N // _TM, 1, _TM), jnp.int32),
            jax.ShapeDtypeStruct((1, 1), jnp.float32),
        ],
    )(z_flat, codebook, zn_row, cn_col)


def _sc_gather(codebook, idx_row):
    mesh = plsc.VectorSubcoreMesh(core_axis_name="core",
                                  subcore_axis_name="subcore")

    @pl.kernel(out_type=jax.ShapeDtypeStruct((_N, _D), jnp.float32),
               mesh=mesh)
    def gather_kernel(cb_hbm, i_hbm, o_hbm):
        def body(i_vmem, o_vmem):
            pltpu.sync_copy(cb_hbm.at[i_vmem.at[0]], o_vmem)

        pltpu.emit_pipeline(
            body,
            grid=(_N // _GATHER_W,),
            in_specs=[pl.BlockSpec((1, _GATHER_W), lambda i: (0, i))],
            out_specs=[pl.BlockSpec((_GATHER_W, _D), lambda i: (i, 0))],
            core_axis_name=("core", "subcore"),
            dimension_semantics=(pltpu.PARALLEL,),
        )(i_hbm, o_hbm)

    return gather_kernel(codebook, idx_row)


def kernel(z, codebook):
    B, C, H, W = z.shape
    z_p = jnp.transpose(z, (0, 2, 3, 1))
    z_flat = z_p.reshape(-1, _D)
    zn = jnp.sum(z_flat ** 2, axis=1, keepdims=True)
    cn = jnp.sum(codebook ** 2, axis=1, keepdims=True)

    z2 = z_flat * jnp.float32(2.0)
    idx_blocks, dsum = _run_argmin(z2, codebook, zn.T, cn)
    idx_row = idx_blocks.reshape(1, _N)

    z_q = _sc_gather(codebook, idx_row)

    m = dsum[0, 0] / jnp.float32(_N * _D)
    codebook_loss = 0.25 * m + m

    quantized = jnp.transpose(z_q.reshape(B, H, W, C), (0, 3, 1, 2))
    indices = idx_row.reshape(B, H, W)
    return (quantized, indices, codebook_loss)


# speedup vs baseline: 1.2161x; 1.0103x over previous
import jax
import jax.numpy as jnp
from jax.experimental import pallas as pl
from jax.experimental.pallas import tpu as pltpu
from jax.experimental.pallas import tpu_sc as plsc

_K = 8192
_D = 256
_N = 8192
_TM = 512
_TN = 1024
_SUB = 8
_GATHER_W = 128


def _argmin_body(z_ref, cb_ref, zn_ref, cn_ref, idx_ref, dsum_ref):
    nblk = _TN // _SUB
    zn = zn_ref[...]

    def chunk_mm(c):
        cb = cb_ref[pl.ds(c * _TN, _TN), :]
        return jax.lax.dot_general(
            cb, z_ref[...], (((1,), (1,)), ((), ())),
            preferred_element_type=jnp.float32)

    def consume(c, mmt, acc, rid):
        cnc = cn_ref[pl.ds(c * _TN, _TN), :]
        for i in range(nblk):
            cni = jax.lax.slice(cnc, (i * _SUB, 0), ((i + 1) * _SUB, 1))
            mmi = jax.lax.slice(mmt, (i * _SUB, 0), ((i + 1) * _SUB, _TM))
            d = (zn + cni) - mmi
            blkid = (c * nblk + i).astype(jnp.float32)
            better = d < acc
            acc = jnp.where(better, d, acc)
            rid = jnp.where(better, blkid, rid)
        return acc, rid

    _PIPE = 8

    def step(jj, carry):
        acc, rid = carry
        ca = jj * _PIPE
        mms = [chunk_mm(ca + p) for p in range(_PIPE)]
        for p in range(_PIPE):
            acc, rid = consume(ca + p, mms[p], acc, rid)
        return acc, rid

    acc0 = jnp.full((_SUB, _TM), jnp.inf, dtype=jnp.float32)
    rid0 = jnp.zeros((_SUB, _TM), dtype=jnp.float32)
    acc, rid = jax.lax.fori_loop(0, _K // _TN // _PIPE, step, (acc0, rid0))

    sub = jax.lax.broadcasted_iota(
        jnp.int32, (_SUB, _TM), 0).astype(jnp.float32)
    absidx = rid * jnp.float32(_SUB) + sub
    gmin = jnp.min(acc, axis=0, keepdims=True)
    cand = jnp.where(acc == gmin, absidx, jnp.float32(_K))
    gidx = jnp.min(cand, axis=0, keepdims=True)

    idx_ref[...] = gidx.astype(jnp.int32).reshape(1, 1, _TM)

    @pl.when(pl.program_id(0) == 0)
    def _():
        dsum_ref[...] = jnp.zeros((1, 1), jnp.float32)
    dsum_ref[...] += jnp.sum(gmin).reshape(1, 1)


def _run_argmin(z_flat, codebook, zn_row, cn_col):
    grid = (_N // _TM,)
    return pl.pallas_call(
        _argmin_body,
        grid=grid,
        in_specs=[
            pl.BlockSpec((_TM, _D), lambda i: (i, 0)),
            pl.BlockSpec((_K, _D), lambda i: (0, 0)),
            pl.BlockSpec((1, _TM), lambda i: (0, i)),
            pl.BlockSpec((_K, 1), lambda i: (0, 0)),
        ],
        out_specs=[
            pl.BlockSpec((1, 1, _TM), lambda i: (i, 0, 0)),
            pl.BlockSpec((1, 1), lambda i: (0, 0)),
        ],
        out_shape=[
            jax.ShapeDtypeStruct((_N // _TM, 1, _TM), jnp.int32),
            jax.ShapeDtypeStruct((1, 1), jnp.float32),
        ],
    )(z_flat, codebook, zn_row, cn_col)


def _sc_gather(codebook, idx_row):
    mesh = plsc.VectorSubcoreMesh(core_axis_name="core",
                                  subcore_axis_name="subcore")

    @pl.kernel(out_type=jax.ShapeDtypeStruct((_N, _D), jnp.float32),
               mesh=mesh)
    def gather_kernel(cb_hbm, i_hbm, o_hbm):
        def body(i_vmem, o_vmem):
            pltpu.sync_copy(cb_hbm.at[i_vmem.at[0]], o_vmem)

        pltpu.emit_pipeline(
            body,
            grid=(_N // _GATHER_W,),
            in_specs=[pl.BlockSpec((1, _GATHER_W), lambda i: (0, i))],
            out_specs=[pl.BlockSpec((_GATHER_W, _D), lambda i: (i, 0))],
            core_axis_name=("core", "subcore"),
            dimension_semantics=(pltpu.PARALLEL,),
        )(i_hbm, o_hbm)

    return gather_kernel(codebook, idx_row)


def kernel(z, codebook):
    B, C, H, W = z.shape
    z_p = jnp.transpose(z, (0, 2, 3, 1))
    z_flat = z_p.reshape(-1, _D)
    zn = jnp.sum(z_flat ** 2, axis=1, keepdims=True)
    cn = jnp.sum(codebook ** 2, axis=1, keepdims=True)

    z2 = z_flat * jnp.float32(2.0)
    idx_blocks, dsum = _run_argmin(z2, codebook, zn.T, cn)
    idx_row = idx_blocks.reshape(1, _N)

    z_q = _sc_gather(codebook, idx_row)

    m = dsum[0, 0] / jnp.float32(_N * _D)
    codebook_loss = 0.25 * m + m

    quantized = jnp.transpose(z_q.reshape(B, H, W, C), (0, 3, 1, 2))
    indices = idx_row.reshape(B, H, W)
    return (quantized, indices, codebook_loss)


# speedup vs baseline: 1.2184x; 1.0019x over previous
import jax
import jax.numpy as jnp
from jax.experimental import pallas as pl
from jax.experimental.pallas import tpu as pltpu
from jax.experimental.pallas import tpu_sc as plsc

_K = 8192
_D = 256
_N = 8192
_TM = 512
_TN = 2048
_SUB = 8
_GATHER_W = 128


def _argmin_body(z_ref, cb_ref, zn_ref, cn_ref, idx_ref, dsum_ref):
    nblk = _TN // _SUB
    zn = zn_ref[...]

    def chunk_mm(c):
        cb = cb_ref[pl.ds(c * _TN, _TN), :]
        return jax.lax.dot_general(
            cb, z_ref[...], (((1,), (1,)), ((), ())),
            preferred_element_type=jnp.float32)

    def consume(c, mmt, acc, rid):
        cnc = cn_ref[pl.ds(c * _TN, _TN), :]
        for i in range(nblk):
            cni = jax.lax.slice(cnc, (i * _SUB, 0), ((i + 1) * _SUB, 1))
            mmi = jax.lax.slice(mmt, (i * _SUB, 0), ((i + 1) * _SUB, _TM))
            d = (zn + cni) - mmi
            blkid = (c * nblk + i).astype(jnp.float32)
            better = d < acc
            acc = jnp.where(better, d, acc)
            rid = jnp.where(better, blkid, rid)
        return acc, rid

    _PIPE = 4

    def step(jj, carry):
        acc, rid = carry
        ca = jj * _PIPE
        mms = [chunk_mm(ca + p) for p in range(_PIPE)]
        for p in range(_PIPE):
            acc, rid = consume(ca + p, mms[p], acc, rid)
        return acc, rid

    acc0 = jnp.full((_SUB, _TM), jnp.inf, dtype=jnp.float32)
    rid0 = jnp.zeros((_SUB, _TM), dtype=jnp.float32)
    acc, rid = jax.lax.fori_loop(0, _K // _TN // _PIPE, step, (acc0, rid0))

    sub = jax.lax.broadcasted_iota(
        jnp.int32, (_SUB, _TM), 0).astype(jnp.float32)
    absidx = rid * jnp.float32(_SUB) + sub
    gmin = jnp.min(acc, axis=0, keepdims=True)
    cand = jnp.where(acc == gmin, absidx, jnp.float32(_K))
    gidx = jnp.min(cand, axis=0, keepdims=True)

    idx_ref[...] = gidx.astype(jnp.int32).reshape(1, 1, _TM)

    @pl.when(pl.program_id(0) == 0)
    def _():
        dsum_ref[...] = jnp.zeros((1, 1), jnp.float32)
    dsum_ref[...] += jnp.sum(gmin).reshape(1, 1)


def _run_argmin(z_flat, codebook, zn_row, cn_col):
    grid = (_N // _TM,)
    return pl.pallas_call(
        _argmin_body,
        grid=grid,
        in_specs=[
            pl.BlockSpec((_TM, _D), lambda i: (i, 0)),
            pl.BlockSpec((_K, _D), lambda i: (0, 0)),
            pl.BlockSpec((1, _TM), lambda i: (0, i)),
            pl.BlockSpec((_K, 1), lambda i: (0, 0)),
        ],
        out_specs=[
            pl.BlockSpec((1, 1, _TM), lambda i: (i, 0, 0)),
            pl.BlockSpec((1, 1), lambda i: (0, 0)),
        ],
        out_shape=[
            jax.ShapeDtypeStruct((_N // _TM, 1, _TM), jnp.int32),
            jax.ShapeDtypeStruct((1, 1), jnp.float32),
        ],
    )(z_flat, codebook, zn_row, cn_col)


def _sc_gather(codebook, idx_row):
    mesh = plsc.VectorSubcoreMesh(core_axis_name="core",
                                  subcore_axis_name="subcore")

    @pl.kernel(out_type=jax.ShapeDtypeStruct((_N, _D), jnp.float32),
               mesh=mesh)
    def gather_kernel(cb_hbm, i_hbm, o_hbm):
        def body(i_vmem, o_vmem):
            pltpu.sync_copy(cb_hbm.at[i_vmem.at[0]], o_vmem)

        pltpu.emit_pipeline(
            body,
            grid=(_N // _GATHER_W,),
            in_specs=[pl.BlockSpec((1, _GATHER_W), lambda i: (0, i))],
            out_specs=[pl.BlockSpec((_GATHER_W, _D), lambda i: (i, 0))],
            core_axis_name=("core", "subcore"),
            dimension_semantics=(pltpu.PARALLEL,),
        )(i_hbm, o_hbm)

    return gather_kernel(codebook, idx_row)


def kernel(z, codebook):
    B, C, H, W = z.shape
    z_p = jnp.transpose(z, (0, 2, 3, 1))
    z_flat = z_p.reshape(-1, _D)
    zn = jnp.sum(z_flat ** 2, axis=1, keepdims=True)
    cn = jnp.sum(codebook ** 2, axis=1, keepdims=True)

    z2 = z_flat * jnp.float32(2.0)
    idx_blocks, dsum = _run_argmin(z2, codebook, zn.T, cn)
    idx_row = idx_blocks.reshape(1, _N)

    z_q = _sc_gather(codebook, idx_row)

    m = dsum[0, 0] / jnp.float32(_N * _D)
    codebook_loss = 0.25 * m + m

    quantized = jnp.transpose(z_q.reshape(B, H, W, C), (0, 3, 1, 2))
    indices = idx_row.reshape(B, H, W)
    return (quantized, indices, codebook_loss)
